# baseline (speedup 1.0000x reference)
import functools

import jax
import jax.numpy as jnp
from jax import lax
from jax.experimental import pallas as pl
from jax.experimental.pallas import tpu as pltpu
from jax.experimental.pallas import tpu_sc as plsc

_NC = 2
_NS = 16
_NW = _NC * _NS
_CHUNK = 128
_GROUP_K = 2


@functools.lru_cache(maxsize=None)
def _make_gather(n_idx, vocab, emb_d):
    assert n_idx % (_NW * _CHUNK) == 0
    n_chunks = n_idx // (_NW * _CHUNK)
    assert n_chunks % _GROUP_K == 0
    n_groups = n_chunks // _GROUP_K
    grp_rows = _GROUP_K * _CHUNK

    @functools.partial(
        pl.kernel,
        out_type=jax.ShapeDtypeStruct((n_idx, emb_d), jnp.float32),
        mesh=plsc.VectorSubcoreMesh(core_axis_name="c", subcore_axis_name="s"),
        compiler_params=pltpu.CompilerParams(use_tc_tiling_on_sc=False),
        scratch_types=[
            pltpu.VMEM((n_chunks, _CHUNK), jnp.int32),
            pltpu.VMEM((2, grp_rows, emb_d), jnp.float32),
            pltpu.SemaphoreType.DMA,
            pltpu.SemaphoreType.DMA,
        ],
    )
    def gather_fn(idx_hbm, table_hbm, out_hbm, idx_v, rows_v, gsem, wsem):
        wid = lax.axis_index("s") * _NC + lax.axis_index("c")
        base_row = wid * n_chunks * _CHUNK
        pltpu.sync_copy(idx_hbm.at[wid], idx_v)

        writebacks = {}
        for g in range(n_groups):
            p = g % 2
            if g >= 2:
                writebacks.pop(g - 2).wait()
            gathers = [
                pltpu.async_copy(
                    table_hbm.at[idx_v.at[g * _GROUP_K + t]],
                    rows_v.at[p, pl.ds(t * _CHUNK, _CHUNK)],
                    gsem,
                )
                for t in range(_GROUP_K)
            ]
            for cp in gathers:
                cp.wait()
            writebacks[g] = pltpu.async_copy(
                rows_v.at[p],
                out_hbm.at[pl.ds(base_row + g * grp_rows, grp_rows)],
                wsem,
            )
        for g in sorted(writebacks):
            writebacks.pop(g).wait()

    return gather_fn


_BMT = 128


@functools.lru_cache(maxsize=None)
def _make_matmul(batch, n_tiles, out_d):
    def body(x_ref, w_ref, b_ref, o_ref):
        acc = None
        for c in range(n_tiles):
            xc = x_ref[:, c].reshape(_BMT * 8, 128)
            wc = w_ref[pl.ds(c * 128, 128), :]
            p = lax.dot_general(
                wc, xc, (((0,), (1,)), ((), ())),
                preferred_element_type=jnp.float32,
            )
            acc = p if acc is None else acc + p
        o_ref[...] = acc + b_ref[...]

    return pl.pallas_call(
        body,
        grid=(batch // (_BMT * 8),),
        in_specs=[
            pl.BlockSpec((_BMT, n_tiles, 8, 128), lambda i: (i, 0, 0, 0)),
            pl.BlockSpec((n_tiles * 128, out_d), lambda i: (0, 0)),
            pl.BlockSpec((out_d, 1), lambda i: (0, 0)),
        ],
        out_specs=pl.BlockSpec((out_d, _BMT * 8), lambda i: (0, i)),
        out_shape=jax.ShapeDtypeStruct((out_d, batch), jnp.float32),
    )


def kernel(sentence, emb, W, b):
    batch, qlen = sentence.shape
    vocab, emb_d = emb.shape
    out_d = W.shape[0]
    n_tiles = (qlen * emb_d + 127) // 128
    q_pad = n_tiles * 128 // emb_d
    n_idx = batch * q_pad

    per_tile = 128 // emb_d
    sp = jnp.pad(sentence, ((0, 0), (0, q_pad - qlen)))
    idx = (
        sp.reshape(batch // 8, 8, n_tiles, per_tile)
        .transpose(0, 2, 1, 3)
        .reshape(_NW, n_idx // (_NW * _CHUNK), _CHUNK)
    )
    gathered = _make_gather(n_idx, vocab, emb_d)(idx, emb)
    x4 = gathered.reshape(batch // 8, n_tiles, 8, 128)
    w_pad = jnp.pad(W.T, ((0, n_tiles * 128 - qlen * emb_d), (0, 0)))
    out_t = _make_matmul(batch, n_tiles, out_d)(x4, w_pad, b.reshape(out_d, 1))
    return out_t.T


# speedup vs baseline: 1.0029x; 1.0029x over previous
import functools

import jax
import jax.numpy as jnp
from jax import lax
from jax.experimental import pallas as pl
from jax.experimental.pallas import tpu as pltpu
from jax.experimental.pallas import tpu_sc as plsc

_NC = 2
_NS = 16
_NW = _NC * _NS
_CHUNK = 128
_GROUP_K = 5


@functools.lru_cache(maxsize=None)
def _make_gather(n_idx, vocab, emb_d):
    assert n_idx % (_NW * _CHUNK) == 0
    n_chunks = n_idx // (_NW * _CHUNK)
    full, rem = divmod(n_chunks, _GROUP_K)
    groups = [_GROUP_K] * full + ([rem] if rem else [])
    grp_rows = _GROUP_K * _CHUNK

    @functools.partial(
        pl.kernel,
        out_type=jax.ShapeDtypeStruct((n_idx, emb_d), jnp.float32),
        mesh=plsc.VectorSubcoreMesh(core_axis_name="c", subcore_axis_name="s"),
        compiler_params=pltpu.CompilerParams(use_tc_tiling_on_sc=False),
        scratch_types=[
            pltpu.VMEM((n_chunks, _CHUNK), jnp.int32),
            pltpu.VMEM((2, grp_rows, emb_d), jnp.float32),
            pltpu.SemaphoreType.DMA,
            pltpu.SemaphoreType.DMA,
        ],
    )
    def gather_fn(idx_hbm, table_hbm, out_hbm, idx_v, rows_v, gsem, wsem):
        wid = lax.axis_index("s") * _NC + lax.axis_index("c")
        base_row = wid * n_chunks * _CHUNK
        pltpu.sync_copy(idx_hbm.at[wid], idx_v)

        writebacks = {}
        chunk0 = 0
        for g, gk in enumerate(groups):
            p = g % 2
            if g >= 2:
                writebacks.pop(g - 2).wait()
            gathers = [
                pltpu.async_copy(
                    table_hbm.at[idx_v.at[chunk0 + t]],
                    rows_v.at[p, pl.ds(t * _CHUNK, _CHUNK)],
                    gsem,
                )
                for t in range(gk)
            ]
            for cp in gathers:
                cp.wait()
            writebacks[g] = pltpu.async_copy(
                rows_v.at[p, pl.ds(0, gk * _CHUNK)],
                out_hbm.at[pl.ds(base_row + chunk0 * _CHUNK, gk * _CHUNK)],
                wsem,
            )
            chunk0 += gk
        for g in sorted(writebacks):
            writebacks.pop(g).wait()

    return gather_fn


_BMT = 128


@functools.lru_cache(maxsize=None)
def _make_matmul(batch, n_tiles, out_d):
    def body(x_ref, w_ref, b_ref, o_ref):
        acc = None
        for c in range(n_tiles):
            xc = x_ref[:, c].reshape(_BMT * 8, 128)
            wc = w_ref[pl.ds(c * 128, 128), :]
            p = lax.dot_general(
                wc, xc, (((0,), (1,)), ((), ())),
                preferred_element_type=jnp.float32,
            )
            acc = p if acc is None else acc + p
        o_ref[...] = acc + b_ref[...]

    return pl.pallas_call(
        body,
        grid=(batch // (_BMT * 8),),
        in_specs=[
            pl.BlockSpec((_BMT, n_tiles, 8, 128), lambda i: (i, 0, 0, 0)),
            pl.BlockSpec((n_tiles * 128, out_d), lambda i: (0, 0)),
            pl.BlockSpec((out_d, 1), lambda i: (0, 0)),
        ],
        out_specs=pl.BlockSpec((out_d, _BMT * 8), lambda i: (0, i)),
        out_shape=jax.ShapeDtypeStruct((out_d, batch), jnp.float32),
    )


def kernel(sentence, emb, W, b):
    batch, qlen = sentence.shape
    vocab, emb_d = emb.shape
    out_d = W.shape[0]
    n_tiles = (qlen * emb_d + 127) // 128
    q_pad = n_tiles * 128 // emb_d
    n_idx = batch * q_pad

    per_tile = 128 // emb_d
    sp = jnp.pad(sentence, ((0, 0), (0, q_pad - qlen)))
    idx = (
        sp.reshape(batch // 8, 8, n_tiles, per_tile)
        .transpose(0, 2, 1, 3)
        .reshape(_NW, n_idx // (_NW * _CHUNK), _CHUNK)
    )
    gathered = _make_gather(n_idx, vocab, emb_d)(idx, emb)
    x4 = gathered.reshape(batch // 8, n_tiles, 8, 128)
    w_pad = jnp.pad(W.T, ((0, n_tiles * 128 - qlen * emb_d), (0, 0)))
    out_t = _make_matmul(batch, n_tiles, out_d)(x4, w_pad, b.reshape(out_d, 1))
    return out_t.T


# speedup vs baseline: 1.5841x; 1.5795x over previous
import functools

import jax
import jax.numpy as jnp
from jax import lax
from jax.experimental import pallas as pl
from jax.experimental.pallas import tpu as pltpu
from jax.experimental.pallas import tpu_sc as plsc

_NC = 2
_NS = 16
_NW = _NC * _NS
_CHUNK = 128
_GROUP_K = 5


@functools.lru_cache(maxsize=None)
def _make_gather(n_idx, vocab, emb_d):
    assert n_idx % (_NW * _CHUNK) == 0
    n_chunks = n_idx // (_NW * _CHUNK)
    full, rem = divmod(n_chunks, _GROUP_K)
    groups = [_GROUP_K] * full + ([rem] if rem else [])
    grp_rows = _GROUP_K * _CHUNK

    @functools.partial(
        pl.kernel,
        out_type=jax.ShapeDtypeStruct((n_idx, emb_d), jnp.float32),
        mesh=plsc.VectorSubcoreMesh(core_axis_name="c", subcore_axis_name="s"),
        compiler_params=pltpu.CompilerParams(use_tc_tiling_on_sc=False),
        scratch_types=[
            pltpu.VMEM((n_chunks, _CHUNK), jnp.int32),
            pltpu.VMEM((2, grp_rows, emb_d), jnp.float32),
            pltpu.SemaphoreType.DMA,
            pltpu.SemaphoreType.DMA,
        ],
    )
    def gather_fn(idx_hbm, table_hbm, out_hbm, idx_v, rows_v, gsem, wsem):
        wid = lax.axis_index("s") * _NC + lax.axis_index("c")
        base_row = wid * n_chunks * _CHUNK
        pltpu.sync_copy(idx_hbm.at[wid], idx_v)

        writebacks = {}
        chunk0 = 0
        for g, gk in enumerate(groups):
            p = g % 2
            if g >= 2:
                writebacks.pop(g - 2).wait()
            gathers = [
                pltpu.async_copy(
                    table_hbm.at[idx_v.at[chunk0 + t]],
                    rows_v.at[p, pl.ds(t * _CHUNK, _CHUNK)],
                    gsem,
                )
                for t in range(gk)
            ]
            for cp in gathers:
                cp.wait()
            writebacks[g] = pltpu.async_copy(
                rows_v.at[p, pl.ds(0, gk * _CHUNK)],
                out_hbm.at[pl.ds(base_row + chunk0 * _CHUNK, gk * _CHUNK)],
                wsem,
            )
            chunk0 += gk
        for g in sorted(writebacks):
            writebacks.pop(g).wait()

    return gather_fn


_BMT = 128


@functools.lru_cache(maxsize=None)
def _make_matmul(batch, n_tiles, out_d):
    def body(x_ref, w_ref, b_ref, o_ref):
        acc = None
        for c in range(n_tiles):
            xc = x_ref[:, c].reshape(_BMT * 8, 128)
            wc = w_ref[pl.ds(c * 128, 128), :]
            p = lax.dot_general(
                wc, xc, (((0,), (1,)), ((), ())),
                preferred_element_type=jnp.float32,
            )
            acc = p if acc is None else acc + p
        o_ref[...] = acc + b_ref[...]

    return pl.pallas_call(
        body,
        grid=(batch // (_BMT * 8),),
        in_specs=[
            pl.BlockSpec((_BMT, n_tiles, 8, 128), lambda i: (i, 0, 0, 0)),
            pl.BlockSpec((n_tiles * 128, out_d), lambda i: (0, 0)),
            pl.BlockSpec((out_d, 1), lambda i: (0, 0)),
        ],
        out_specs=pl.BlockSpec((out_d, _BMT * 8), lambda i: (0, i)),
        out_shape=jax.ShapeDtypeStruct((out_d, batch), jnp.float32),
    )


def kernel(sentence, emb, W, b):
    batch, qlen = sentence.shape
    vocab, emb_d = emb.shape
    out_d = W.shape[0]
    n_tiles = (qlen * emb_d + 127) // 128
    q_pad = n_tiles * 128 // emb_d
    n_idx = batch * q_pad

    per_tile = 128 // emb_d
    sp = jnp.pad(sentence, ((0, 0), (0, q_pad - qlen)), mode="edge")
    idx = (
        sp.reshape(batch // 8, 8, n_tiles, per_tile)
        .transpose(0, 2, 1, 3)
        .reshape(_NW, n_idx // (_NW * _CHUNK), _CHUNK)
    )
    gathered = _make_gather(n_idx, vocab, emb_d)(idx, emb)
    x4 = gathered.reshape(batch // 8, n_tiles, 8, 128)
    w_pad = jnp.pad(W.T, ((0, n_tiles * 128 - qlen * emb_d), (0, 0)))
    out_t = _make_matmul(batch, n_tiles, out_d)(x4, w_pad, b.reshape(out_d, 1))
    return out_t.T


# speedup vs baseline: 2.0644x; 1.3033x over previous
import functools

import numpy as np

import jax
import jax.numpy as jnp
from jax import lax
from jax.experimental import pallas as pl
from jax.experimental.pallas import tpu as pltpu
from jax.experimental.pallas import tpu_sc as plsc

_NC = 2
_NS = 16
_NW = _NC * _NS
_L = 16
_CHUNK = 128
_GROUP_K = 5


def _tile_perm(qlen, batch, n_tiles, per_tile):
    n_idx = (batch // 8) * n_tiles * 8 * per_tile
    n = np.arange(n_idx, dtype=np.int64)
    j = n % per_tile
    r = (n // per_tile) % 8
    c = (n // (per_tile * 8)) % n_tiles
    b0 = n // (per_tile * 8 * n_tiles)
    q = per_tile * c + j
    q = np.where(q >= qlen, 0, q)
    pos = q * batch + 8 * b0 + r
    return pos.astype(np.int32).reshape(_NW, -1, _CHUNK)


@functools.lru_cache(maxsize=None)
def _make_gather(qlen, batch, vocab, emb_d, n_tiles):
    per_tile = 128 // emb_d
    n_local = (batch // _NW) * n_tiles * per_tile
    assert n_local % _CHUNK == 0
    n_chunks = n_local // _CHUNK
    n_idx = n_local * _NW
    full, rem = divmod(n_chunks, _GROUP_K)
    groups = [_GROUP_K] * full + ([rem] if rem else [])
    grp_rows = _GROUP_K * _CHUNK
    pos_const = _tile_perm(qlen, batch, n_tiles, per_tile)

    @functools.partial(
        pl.kernel,
        out_type=jax.ShapeDtypeStruct((n_idx, emb_d), jnp.float32),
        mesh=plsc.VectorSubcoreMesh(core_axis_name="c", subcore_axis_name="s"),
        compiler_params=pltpu.CompilerParams(use_tc_tiling_on_sc=False),
        scratch_types=[
            pltpu.VMEM((n_chunks, _CHUNK), jnp.int32),
            pltpu.VMEM((n_local,), jnp.int32),
            pltpu.VMEM((2, grp_rows, emb_d), jnp.float32),
            pltpu.SemaphoreType.DMA,
            pltpu.SemaphoreType.DMA,
            pltpu.SemaphoreType.DMA,
        ],
    )
    def gather_fn(sent1_hbm, pos_hbm, table_hbm, out_hbm,
                  pos_v, idx_v, rows_v, psem, gsem, wsem):
        wid = lax.axis_index("s") * _NC + lax.axis_index("c")
        base_row = wid * n_local
        pltpu.sync_copy(pos_hbm.at[wid], pos_v)

        fetches = [
            pltpu.async_copy(
                sent1_hbm.at[pos_v.at[cc]],
                idx_v.at[pl.ds(cc * _CHUNK, _CHUNK)],
                psem,
            )
            for cc in range(n_chunks)
        ]
        for cp in fetches:
            cp.wait()

        writebacks = {}
        chunk0 = 0
        for g, gk in enumerate(groups):
            p = g % 2
            if g >= 2:
                writebacks.pop(g - 2).wait()
            gathers = [
                pltpu.async_copy(
                    table_hbm.at[idx_v.at[pl.ds((chunk0 + t) * _CHUNK, _CHUNK)]],
                    rows_v.at[p, pl.ds(t * _CHUNK, _CHUNK)],
                    gsem,
                )
                for t in range(gk)
            ]
            for cp in gathers:
                cp.wait()
            writebacks[g] = pltpu.async_copy(
                rows_v.at[p, pl.ds(0, gk * _CHUNK)],
                out_hbm.at[pl.ds(base_row + chunk0 * _CHUNK, gk * _CHUNK)],
                wsem,
            )
            chunk0 += gk
        for g in sorted(writebacks):
            writebacks.pop(g).wait()

    def run(sent1, emb):
        pos = jnp.asarray(pos_const)
        return gather_fn(sent1, pos, emb)

    return run


_BMT = 128


@functools.lru_cache(maxsize=None)
def _make_matmul(batch, n_tiles, out_d):
    def body(x_ref, w_ref, b_ref, o_ref):
        acc = None
        for c in range(n_tiles):
            xc = x_ref[:, c].reshape(_BMT * 8, 128)
            wc = w_ref[pl.ds(c * 128, 128), :]
            p = lax.dot_general(
                wc, xc, (((0,), (1,)), ((), ())),
                preferred_element_type=jnp.float32,
            )
            acc = p if acc is None else acc + p
        o_ref[...] = acc + b_ref[...]

    return pl.pallas_call(
        body,
        grid=(batch // (_BMT * 8),),
        in_specs=[
            pl.BlockSpec((_BMT, n_tiles, 8, 128), lambda i: (i, 0, 0, 0)),
            pl.BlockSpec((n_tiles * 128, out_d), lambda i: (0, 0)),
            pl.BlockSpec((out_d, 1), lambda i: (0, 0)),
        ],
        out_specs=pl.BlockSpec((out_d, _BMT * 8), lambda i: (0, i)),
        out_shape=jax.ShapeDtypeStruct((out_d, batch), jnp.float32),
    )


def kernel(sentence, emb, W, b):
    batch, qlen = sentence.shape
    vocab, emb_d = emb.shape
    out_d = W.shape[0]
    n_tiles = (qlen * emb_d + 127) // 128

    sent1 = sentence.T.reshape(-1)
    gathered = _make_gather(qlen, batch, vocab, emb_d, n_tiles)(sent1, emb)
    x4 = gathered.reshape(batch // 8, n_tiles, 8, 128)
    w_pad = jnp.pad(W.T, ((0, n_tiles * 128 - qlen * emb_d), (0, 0)))
    out_t = _make_matmul(batch, n_tiles, out_d)(x4, w_pad, b.reshape(out_d, 1))
    return out_t.T
